# Initial kernel scaffold; baseline (speedup 1.0000x reference)
import jax
import jax.numpy as jnp
from jax.experimental import pallas as pl


def kernel(des, tweet, num_prop, cat_prop, edge_index, edge_type, W_des, b_des, W_tw, b_tw, W_np, b_np, W_cp, b_cp, W_in, b_in, Wg1, bg1, Wg2, bg2, W_o1, b_o1, W_o2, b_o2):
    raise NotImplementedError("write your pallas kernel here")


# speedup vs baseline: 1.0151x; 1.0151x over previous
import functools

import jax
import jax.numpy as jnp
from jax.experimental import pallas as pl
from jax.experimental.pallas import tpu as pltpu


def _leaky(x):
    return jnp.where(x >= 0, x, 0.01 * x)


_RB = 1000


def _fusion_body(des_ref, tw_ref, np_ref, cp_ref,
                 wd_ref, bd_ref, wt_ref, bt_ref, wn_ref, bn_ref,
                 wc_ref, bc_ref, wi_ref, bi_ref, o_ref):
    d = _leaky(jnp.dot(des_ref[...], wd_ref[...],
                       preferred_element_type=jnp.float32) + bd_ref[...])
    t = _leaky(jnp.dot(tw_ref[...], wt_ref[...],
                       preferred_element_type=jnp.float32) + bt_ref[...])
    n = _leaky(jnp.dot(np_ref[...], wn_ref[...],
                       preferred_element_type=jnp.float32) + bn_ref[...])
    c = _leaky(jnp.dot(cp_ref[...], wc_ref[...],
                       preferred_element_type=jnp.float32) + bc_ref[...])
    x = jnp.concatenate([d, t, n, c], axis=1)
    o_ref[...] = _leaky(jnp.dot(x, wi_ref[...],
                                preferred_element_type=jnp.float32) + bi_ref[...])


def _fusion(des, tweet, num_prop, cat_prop,
            W_des, b_des, W_tw, b_tw, W_np, b_np, W_cp, b_cp, W_in, b_in):
    n = des.shape[0]
    q = W_des.shape[1]
    d = W_in.shape[1]
    row = lambda i: (i, 0)
    full = lambda i: (0, 0)
    return pl.pallas_call(
        _fusion_body,
        grid=(n // _RB,),
        in_specs=[
            pl.BlockSpec((_RB, des.shape[1]), row),
            pl.BlockSpec((_RB, tweet.shape[1]), row),
            pl.BlockSpec((_RB, num_prop.shape[1]), row),
            pl.BlockSpec((_RB, cat_prop.shape[1]), row),
            pl.BlockSpec(W_des.shape, full), pl.BlockSpec((1, q), full),
            pl.BlockSpec(W_tw.shape, full), pl.BlockSpec((1, q), full),
            pl.BlockSpec(W_np.shape, full), pl.BlockSpec((1, q), full),
            pl.BlockSpec(W_cp.shape, full), pl.BlockSpec((1, q), full),
            pl.BlockSpec(W_in.shape, full), pl.BlockSpec((1, d), full),
        ],
        out_specs=pl.BlockSpec((_RB, d), row),
        out_shape=jax.ShapeDtypeStruct((n, d), jnp.float32),
    )(des, tweet, num_prop, cat_prop,
      W_des, b_des.reshape(1, -1), W_tw, b_tw.reshape(1, -1),
      W_np, b_np.reshape(1, -1), W_cp, b_cp.reshape(1, -1),
      W_in, b_in.reshape(1, -1))


def _head_body(x_ref, w1_ref, b1_ref, w2_ref, b2_ref, o_ref):
    h = _leaky(jnp.dot(x_ref[...], w1_ref[...],
                       preferred_element_type=jnp.float32) + b1_ref[...])
    o_ref[...] = jnp.dot(h, w2_ref[...],
                         preferred_element_type=jnp.float32) + b2_ref[...]


def _head(x, W_o1, b_o1, W_o2, b_o2):
    n, d = x.shape
    k = W_o2.shape[1]
    return pl.pallas_call(
        _head_body,
        grid=(n // _RB,),
        in_specs=[
            pl.BlockSpec((_RB, d), lambda i: (i, 0)),
            pl.BlockSpec(W_o1.shape, lambda i: (0, 0)),
            pl.BlockSpec((1, d), lambda i: (0, 0)),
            pl.BlockSpec(W_o2.shape, lambda i: (0, 0)),
            pl.BlockSpec((1, k), lambda i: (0, 0)),
        ],
        out_specs=pl.BlockSpec((_RB, k), lambda i: (i, 0)),
        out_shape=jax.ShapeDtypeStruct((n, k), jnp.float32),
    )(x, W_o1, b_o1.reshape(1, -1), W_o2, b_o2.reshape(1, -1))


def _gcn_conv(x, edge_index, W, b):
    n = x.shape[0]
    loop = jnp.arange(n, dtype=edge_index.dtype)
    src = jnp.concatenate([edge_index[0], loop])
    dst = jnp.concatenate([edge_index[1], loop])
    deg = jnp.zeros((n,), x.dtype).at[dst].add(jnp.ones((src.shape[0],), x.dtype))
    dinv = jnp.where(deg > 0, 1.0 / jnp.sqrt(deg), 0.0)
    norm = dinv[src] * dinv[dst]
    h = x @ W
    msg = h[src] * norm[:, None]
    out = jnp.zeros((n, W.shape[1]), x.dtype).at[dst].add(msg)
    return out + b


def kernel(des, tweet, num_prop, cat_prop, edge_index, edge_type,
           W_des, b_des, W_tw, b_tw, W_np, b_np, W_cp, b_cp,
           W_in, b_in, Wg1, bg1, Wg2, bg2, W_o1, b_o1, W_o2, b_o2):
    x = _fusion(des, tweet, num_prop, cat_prop,
                W_des, b_des, W_tw, b_tw, W_np, b_np, W_cp, b_cp, W_in, b_in)
    x = _gcn_conv(x, edge_index, Wg1, bg1)
    x = _gcn_conv(x, edge_index, Wg2, bg2)
    return _head(x, W_o1, b_o1, W_o2, b_o2)


# speedup vs baseline: 18.7629x; 18.4845x over previous
import functools

import jax
import jax.numpy as jnp
from jax import lax
from jax.experimental import pallas as pl
from jax.experimental.pallas import tpu as pltpu
from jax.experimental.pallas import tpu_sc as plsc


def _leaky(x):
    return jnp.where(x >= 0, x, 0.01 * x)


_RB = 1000
_CH = 128
_NC = 2
_NS = 16


def _fusion_body(des_ref, tw_ref, np_ref, cp_ref,
                 wd_ref, bd_ref, wt_ref, bt_ref, wn_ref, bn_ref,
                 wc_ref, bc_ref, wi_ref, bi_ref, o_ref):
    d = _leaky(jnp.dot(des_ref[...], wd_ref[...],
                       preferred_element_type=jnp.float32) + bd_ref[...])
    t = _leaky(jnp.dot(tw_ref[...], wt_ref[...],
                       preferred_element_type=jnp.float32) + bt_ref[...])
    n = _leaky(jnp.dot(np_ref[...], wn_ref[...],
                       preferred_element_type=jnp.float32) + bn_ref[...])
    c = _leaky(jnp.dot(cp_ref[...], wc_ref[...],
                       preferred_element_type=jnp.float32) + bc_ref[...])
    x = jnp.concatenate([d, t, n, c], axis=1)
    o_ref[...] = _leaky(jnp.dot(x, wi_ref[...],
                                preferred_element_type=jnp.float32) + bi_ref[...])


def _fusion(des, tweet, num_prop, cat_prop,
            W_des, b_des, W_tw, b_tw, W_np, b_np, W_cp, b_cp, W_in, b_in):
    n = des.shape[0]
    q = W_des.shape[1]
    d = W_in.shape[1]
    row = lambda i: (i, 0)
    full = lambda i: (0, 0)
    return pl.pallas_call(
        _fusion_body,
        grid=(n // _RB,),
        in_specs=[
            pl.BlockSpec((_RB, des.shape[1]), row),
            pl.BlockSpec((_RB, tweet.shape[1]), row),
            pl.BlockSpec((_RB, num_prop.shape[1]), row),
            pl.BlockSpec((_RB, cat_prop.shape[1]), row),
            pl.BlockSpec(W_des.shape, full), pl.BlockSpec((1, q), full),
            pl.BlockSpec(W_tw.shape, full), pl.BlockSpec((1, q), full),
            pl.BlockSpec(W_np.shape, full), pl.BlockSpec((1, q), full),
            pl.BlockSpec(W_cp.shape, full), pl.BlockSpec((1, q), full),
            pl.BlockSpec(W_in.shape, full), pl.BlockSpec((1, d), full),
        ],
        out_specs=pl.BlockSpec((_RB, d), row),
        out_shape=jax.ShapeDtypeStruct((n, d), jnp.float32),
    )(des, tweet, num_prop, cat_prop,
      W_des, b_des.reshape(1, -1), W_tw, b_tw.reshape(1, -1),
      W_np, b_np.reshape(1, -1), W_cp, b_cp.reshape(1, -1),
      W_in, b_in.reshape(1, -1))


def _dinv_of(degp):
    return lax.rsqrt(degp[0] + degp[1] + 1.0)


def _pre_body(x_ref, w_ref, degp_ref, o_ref):
    h = jnp.dot(x_ref[...], w_ref[...], preferred_element_type=jnp.float32)
    g = _dinv_of(degp_ref[...]) * h
    d2 = g.shape[1] // 2
    o_ref[0] = g[:, :d2]
    o_ref[1] = g[:, d2:]


def _mid_body(s_ref, g_ref, degp_ref, b_ref, w_ref, o_ref):
    dinv = _dinv_of(degp_ref[...])
    s = jnp.concatenate([s_ref[0], s_ref[1]], axis=1)
    g = jnp.concatenate([g_ref[0], g_ref[1]], axis=1)
    x1 = dinv * (s + g) + b_ref[...]
    g2 = dinv * jnp.dot(x1, w_ref[...], preferred_element_type=jnp.float32)
    d2 = g2.shape[1] // 2
    o_ref[0] = g2[:, :d2]
    o_ref[1] = g2[:, d2:]


def _post_body(s_ref, g_ref, degp_ref, b_ref, w1_ref, b1_ref, w2_ref, b2_ref,
               o_ref):
    dinv = _dinv_of(degp_ref[...])
    s = jnp.concatenate([s_ref[0], s_ref[1]], axis=1)
    g = jnp.concatenate([g_ref[0], g_ref[1]], axis=1)
    x2 = dinv * (s + g) + b_ref[...]
    h = _leaky(jnp.dot(x2, w1_ref[...],
                       preferred_element_type=jnp.float32) + b1_ref[...])
    o_ref[...] = jnp.dot(h, w2_ref[...],
                         preferred_element_type=jnp.float32) + b2_ref[...]


def _sc_meshes():
    return plsc.VectorSubcoreMesh(core_axis_name="c", subcore_axis_name="s")


def _deg_call(dst2, nacc):
    rows = dst2.shape[0]
    rpt = rows // (_NC * _NS)
    grp = 4
    zcp = nacc // _NS // _CH

    @functools.partial(
        pl.kernel,
        out_type=jax.ShapeDtypeStruct((_NC, nacc), jnp.float32),
        mesh=_sc_meshes(),
        compiler_params=pltpu.CompilerParams(use_tc_tiling_on_sc=False),
        scratch_types=[
            pltpu.VMEM_SHARED((nacc,), jnp.float32),
            pltpu.VMEM((grp, _CH), jnp.int32),
            pltpu.VMEM((_CH,), jnp.float32),
            pltpu.VMEM((_CH,), jnp.float32),
        ],
    )
    def deg_kernel(dst2_hbm, degp_hbm, acc, idxb, onesb, zb):
        c = lax.axis_index("c")
        s = lax.axis_index("s")

        def fill(i, _):
            onesb[pl.ds(i * 16, 16)] = jnp.full((16,), 1.0, jnp.float32)
            zb[pl.ds(i * 16, 16)] = jnp.zeros((16,), jnp.float32)
            return 0
        lax.fori_loop(0, _CH // 16, fill, 0)

        def zero(i, _):
            pltpu.sync_copy(zb, acc.at[pl.ds((s * zcp + i) * _CH, _CH)])
            return 0
        lax.fori_loop(0, zcp, zero, 0)
        plsc.subcore_barrier()

        base = (c * _NS + s) * rpt

        def body(gi, _):
            row0 = base + gi * grp
            pltpu.sync_copy(dst2_hbm.at[pl.ds(row0, grp)], idxb)
            for j in range(grp):
                pltpu.sync_copy(onesb, acc.at[idxb.at[j]], add=True)
            return 0
        lax.fori_loop(0, rpt // grp, body, 0)
        plsc.subcore_barrier()

        def out(i, _):
            off = (s * zcp + i) * _CH
            pltpu.sync_copy(acc.at[pl.ds(off, _CH)],
                            degp_hbm.at[c, pl.ds(off, _CH)])
            return 0
        lax.fori_loop(0, zcp, out, 0)

    return deg_kernel(dst2)


def _agg_call(sr3, dst2, gflat, nacc):
    rows = dst2.shape[0]
    d2 = gflat.shape[1]
    rpt = rows // _NS
    grp = 4
    zcp = nacc // _NS // _CH

    @functools.partial(
        pl.kernel,
        out_type=jax.ShapeDtypeStruct((_NC, nacc, d2), jnp.float32),
        mesh=_sc_meshes(),
        compiler_params=pltpu.CompilerParams(use_tc_tiling_on_sc=False),
        scratch_types=[
            pltpu.VMEM_SHARED((nacc, d2), jnp.float32),
            pltpu.VMEM((grp, _CH), jnp.int32),
            pltpu.VMEM((grp, _CH), jnp.int32),
            pltpu.VMEM((grp, _CH, d2), jnp.float32),
            pltpu.SemaphoreType.DMA,
        ],
    )
    def agg_kernel(sr3_hbm, dst2_hbm, g_hbm, sout_hbm,
                   acc, sbuf, dbuf, rbuf, gsem):
        c = lax.axis_index("c")
        s = lax.axis_index("s")

        def zfill(i, _):
            for k in range(d2 // 16):
                rbuf[0, i, pl.ds(k * 16, 16)] = jnp.zeros((16,), jnp.float32)
            return 0
        lax.fori_loop(0, _CH, zfill, 0)

        def zero(i, _):
            pltpu.sync_copy(rbuf.at[0], acc.at[pl.ds((s * zcp + i) * _CH, _CH)])
            return 0
        lax.fori_loop(0, zcp, zero, 0)
        plsc.subcore_barrier()

        base = s * rpt

        def body(gi, _):
            row0 = base + gi * grp
            pltpu.sync_copy(sr3_hbm.at[c, pl.ds(row0, grp)], sbuf)
            pltpu.sync_copy(dst2_hbm.at[pl.ds(row0, grp)], dbuf)
            cps = [pltpu.async_copy(g_hbm.at[sbuf.at[j]], rbuf.at[j], gsem)
                   for j in range(grp)]
            for cp in cps:
                cp.wait()
            for j in range(grp):
                pltpu.sync_copy(rbuf.at[j], acc.at[dbuf.at[j]], add=True)
            return 0
        lax.fori_loop(0, rpt // grp, body, 0)
        plsc.subcore_barrier()

        def out(i, _):
            off = (s * zcp + i) * _CH
            pltpu.sync_copy(acc.at[pl.ds(off, _CH)],
                            sout_hbm.at[c, pl.ds(off, _CH)])
            return 0
        lax.fori_loop(0, zcp, out, 0)

    return agg_kernel(sr3, dst2, gflat)


def kernel(des, tweet, num_prop, cat_prop, edge_index, edge_type,
           W_des, b_des, W_tw, b_tw, W_np, b_np, W_cp, b_cp,
           W_in, b_in, Wg1, bg1, Wg2, bg2, W_o1, b_o1, W_o2, b_o2):
    n = des.shape[0]
    e = edge_index.shape[1]
    d = W_in.shape[1]
    d2 = d // 2

    unit = _CH * _NC * _NS
    pe = ((e + unit - 1) // unit) * unit
    pad = pe - e
    garbage = 1200
    nacc = ((n + garbage + _NS * _CH - 1) // (_NS * _CH)) * (_NS * _CH)
    src = jnp.concatenate(
        [edge_index[0], jnp.zeros((pad,), edge_index.dtype)])
    dst = jnp.concatenate(
        [edge_index[1],
         n + (jnp.arange(pad, dtype=edge_index.dtype) % garbage)])
    rows = pe // _CH
    sr3 = jnp.stack([src, src + n]).reshape(_NC, rows, _CH)
    dst2 = dst.reshape(rows, _CH)

    degp = _deg_call(dst2, nacc).reshape(_NC, nacc, 1)

    x = _fusion(des, tweet, num_prop, cat_prop,
                W_des, b_des, W_tw, b_tw, W_np, b_np, W_cp, b_cp, W_in, b_in)

    row = lambda i: (i, 0)
    full = lambda i: (0, 0)
    stk = lambda i: (0, i, 0)
    spec_half = pl.BlockSpec((_NC, _RB, d2), stk)
    spec_deg = pl.BlockSpec((_NC, _RB, 1), stk)
    grid = (n // _RB,)

    g1 = pl.pallas_call(
        _pre_body,
        grid=grid,
        in_specs=[pl.BlockSpec((_RB, d), row), pl.BlockSpec(Wg1.shape, full),
                  spec_deg],
        out_specs=spec_half,
        out_shape=jax.ShapeDtypeStruct((_NC, n, d2), jnp.float32),
    )(x, Wg1, degp)

    s1 = _agg_call(sr3, dst2, g1.reshape(_NC * n, d2), nacc)

    g2 = pl.pallas_call(
        _mid_body,
        grid=grid,
        in_specs=[spec_half, spec_half, spec_deg,
                  pl.BlockSpec((1, d), full), pl.BlockSpec(Wg2.shape, full)],
        out_specs=spec_half,
        out_shape=jax.ShapeDtypeStruct((_NC, n, d2), jnp.float32),
    )(s1, g1, degp, bg1.reshape(1, -1), Wg2)

    s2 = _agg_call(sr3, dst2, g2.reshape(_NC * n, d2), nacc)

    out = pl.pallas_call(
        _post_body,
        grid=grid,
        in_specs=[spec_half, spec_half, spec_deg,
                  pl.BlockSpec((1, d), full),
                  pl.BlockSpec(W_o1.shape, full), pl.BlockSpec((1, d), full),
                  pl.BlockSpec(W_o2.shape, full),
                  pl.BlockSpec((1, W_o2.shape[1]), full)],
        out_specs=pl.BlockSpec((_RB, W_o2.shape[1]), row),
        out_shape=jax.ShapeDtypeStruct((n, W_o2.shape[1]), jnp.float32),
    )(s2, g2, degp, bg2.reshape(1, -1),
      W_o1, b_o1.reshape(1, -1), W_o2, b_o2.reshape(1, -1))

    return out
